# Initial kernel scaffold; baseline (speedup 1.0000x reference)
import jax
import jax.numpy as jnp
from jax.experimental import pallas as pl


def kernel(V, E, X, W, b):
    raise NotImplementedError("write your pallas kernel here")


# speedup vs baseline: 12.4823x; 12.4823x over previous
import functools

import jax
import jax.numpy as jnp
from jax import lax
from jax.experimental import pallas as pl
from jax.experimental.pallas import tpu as pltpu
from jax.experimental.pallas import tpu_sc as plsc

NC = 2
NS = 16
LANES = 16
NW = NC * NS

_MESH = plsc.VectorSubcoreMesh(core_axis_name="c", subcore_axis_name="s")
_SC_PARAMS = pltpu.CompilerParams(needs_layout_passes=False)


def _make_deg_kernel(NP, EPT):
    @functools.partial(
        pl.kernel,
        mesh=_MESH,
        compiler_params=_SC_PARAMS,
        out_type=jax.ShapeDtypeStruct((NW, NP), jnp.float32),
        scratch_types=[
            pltpu.VMEM((EPT,), jnp.int32),
            pltpu.VMEM((NP,), jnp.float32),
        ],
    )
    def deg_kernel(col_hbm, z1_hbm, degp_hbm, col_v, hist_v):
        c = lax.axis_index("c")
        s = lax.axis_index("s")
        wid = c * NS + s
        pltpu.sync_copy(z1_hbm, hist_v)
        pltpu.sync_copy(col_hbm.at[wid], col_v)
        ones16 = jnp.full((LANES,), 1.0, jnp.float32)

        def body(i, carry):
            idx = col_v[pl.ds(i * LANES, LANES)]
            plsc.addupdate_scatter(hist_v, [idx], ones16)
            return carry

        lax.fori_loop(0, EPT // LANES, body, 0)
        pltpu.sync_copy(hist_v, degp_hbm.at[wid])

    return deg_kernel


def _make_hop_kernel(NP, D, CH):
    RPT = NP // NS

    @functools.partial(
        pl.kernel,
        mesh=_MESH,
        compiler_params=_SC_PARAMS,
        out_type=jax.ShapeDtypeStruct((NC, NP, D), jnp.float32),
        scratch_types=[
            pltpu.VMEM((CH, 128), jnp.int32),
            pltpu.VMEM((CH, 128), jnp.int32),
            pltpu.VMEM((128, D), jnp.float32),
            pltpu.VMEM_SHARED((NP, D), jnp.float32),
            pltpu.SemaphoreType.DMA,
        ],
    )
    def hop_kernel(g_hbm, row_hbm, col_hbm, z2_hbm, p_hbm,
                   row_v, col_v, rows_v, s_sh, sem):
        c = lax.axis_index("c")
        s = lax.axis_index("s")
        wid = c * NS + s
        pltpu.sync_copy(z2_hbm, s_sh.at[pl.ds(s * RPT, RPT)])
        pltpu.sync_copy(row_hbm.at[wid], row_v)
        pltpu.sync_copy(col_hbm.at[wid], col_v)
        plsc.subcore_barrier()

        def body(j, carry):
            pltpu.async_copy(g_hbm.at[row_v.at[j]], rows_v, sem).wait()
            pltpu.sync_copy(rows_v, s_sh.at[col_v.at[j]], add=True)
            return carry

        lax.fori_loop(0, CH, body, 0)
        plsc.subcore_barrier()
        pltpu.sync_copy(s_sh.at[pl.ds(s * RPT, RPT)],
                        p_hbm.at[c, pl.ds(s * RPT, RPT)])

    return hop_kernel


def _prep_body(degp_ref, x_ref, g0_ref, d_ref):
    deg = jnp.sum(degp_ref[...], axis=0) + 1.0
    d = lax.rsqrt(deg)
    d_ref[...] = d
    g0_ref[...] = d[:, None] * x_ref[...]


def _combine_body(p0_ref, p1_ref, g_ref, d_ref, out_ref):
    d = d_ref[...]
    out_ref[...] = (d * d)[:, None] * (p0_ref[...] + p1_ref[...] + g_ref[...])


def _final_body(p0_ref, p1_ref, g_ref, d_ref, w_ref, b_ref, out_ref):
    t = d_ref[...][:, None] * (p0_ref[...] + p1_ref[...] + g_ref[...])
    out_ref[...] = (
        jnp.dot(t, w_ref[...], preferred_element_type=jnp.float32)
        + b_ref[...]
    )


def kernel(V, E, X, W, b):
    N, D = X.shape
    DO = W.shape[1]
    E0 = E.shape[1]

    NP = ((N + 1023) // 1024) * 1024
    EPT = ((E0 + NW * 128 - 1) // (NW * 128)) * 128
    EP = EPT * NW
    CH = EPT // 128

    pad_e = EP - E0
    rowp = jnp.concatenate([E[0], jnp.zeros((pad_e,), jnp.int32)])
    colp = jnp.concatenate([E[1], jnp.full((pad_e,), NP - 1, jnp.int32)])
    row3 = rowp.reshape(NW, CH, 128)
    col3 = colp.reshape(NW, CH, 128)
    colflat = colp.reshape(NW, EPT)
    Xp = jnp.pad(X, ((0, NP - N), (0, 0)))
    z1 = jnp.zeros((NP,), jnp.float32)
    z2 = jnp.zeros((NP // NS, D), jnp.float32)

    deg_kernel = _make_deg_kernel(NP, EPT)
    hop_kernel = _make_hop_kernel(NP, D, CH)

    degp = deg_kernel(colflat, z1)

    BR = 1024
    grid = (NP // BR,)
    g0, dvec = pl.pallas_call(
        _prep_body,
        grid=grid,
        in_specs=[
            pl.BlockSpec((NW, BR), lambda i: (0, i)),
            pl.BlockSpec((BR, D), lambda i: (i, 0)),
        ],
        out_specs=[
            pl.BlockSpec((BR, D), lambda i: (i, 0)),
            pl.BlockSpec((BR,), lambda i: (i,)),
        ],
        out_shape=[
            jax.ShapeDtypeStruct((NP, D), jnp.float32),
            jax.ShapeDtypeStruct((NP,), jnp.float32),
        ],
    )(degp, Xp)

    p = hop_kernel(g0, row3, col3, z2)

    g1 = pl.pallas_call(
        _combine_body,
        grid=grid,
        in_specs=[
            pl.BlockSpec((BR, D), lambda i: (i, 0)),
            pl.BlockSpec((BR, D), lambda i: (i, 0)),
            pl.BlockSpec((BR, D), lambda i: (i, 0)),
            pl.BlockSpec((BR,), lambda i: (i,)),
        ],
        out_specs=pl.BlockSpec((BR, D), lambda i: (i, 0)),
        out_shape=jax.ShapeDtypeStruct((NP, D), jnp.float32),
    )(p[0], p[1], g0, dvec)

    p2 = hop_kernel(g1, row3, col3, z2)

    out = pl.pallas_call(
        _final_body,
        grid=grid,
        in_specs=[
            pl.BlockSpec((BR, D), lambda i: (i, 0)),
            pl.BlockSpec((BR, D), lambda i: (i, 0)),
            pl.BlockSpec((BR, D), lambda i: (i, 0)),
            pl.BlockSpec((BR,), lambda i: (i,)),
            pl.BlockSpec((D, DO), lambda i: (0, 0)),
            pl.BlockSpec((1, DO), lambda i: (0, 0)),
        ],
        out_specs=pl.BlockSpec((BR, DO), lambda i: (i, 0)),
        out_shape=jax.ShapeDtypeStruct((NP, DO), jnp.float32),
    )(p2[0], p2[1], g1, dvec, W, b.reshape(1, DO))

    return out[:N]
